# baseline (speedup 1.0000x reference)
import functools

import jax
import jax.numpy as jnp
from jax import lax
from jax.experimental import pallas as pl
from jax.experimental.pallas import tpu as pltpu
from jax.experimental.pallas import tpu_sc as plsc

K = 1024
L = 16
NIMG = 8
NPIX = 512 * 512
QUART = NPIX // 4
CHUNK = 8192
HSIZE = 4 * L * K
ROWS = 4 * K


def _body(x_hbm, t_hbm, out_hbm, xbuf, tbuf, xbuf2, tbuf2, hist, hist2, tmp2,
          obuf, shared, semx0, semt0, semx1, semt1):
    c = lax.axis_index("c")
    s = lax.axis_index("s")
    img = c * 4 + s // 4
    q = s % 4

    lane_off = lax.iota(jnp.int32, 16) * K
    zero16 = jnp.zeros((16,), jnp.float32)
    ones16 = jnp.ones((16,), jnp.float32)


    G = 32
    amax = jnp.float32(1.0 - 2e-7)

    def make_ibody(xb, tb):
        ipr = 512 // (G * 16)

        def ibody(i, carry):
            row = i // ipr
            cb = (i % ipr) * (G * 16)
            xs = [xb[row, pl.ds(cb + g * 16, 16)] for g in range(G)]
            ts = [tb[row, pl.ds(cb + g * 16, 16)] for g in range(G)]
            es = [jnp.exp(-x) for x in xs]
            ps = [1.0 / (1.0 + e) for e in es]
            us = [p + p - 1.0 for p in ps]
            avs = [jnp.minimum(jnp.abs(u), amax) for u in us]
            bs = [((u + 1.0) * (K * 0.5)).astype(jnp.int32) for u in us]
            bs = [jnp.minimum(b, K - 1) for b in bs]
            idxs = [(t << 14) | lane_off | b for t, b in zip(ts, bs)]
            for idx in idxs:
                plsc.addupdate_scatter(hist, [idx], ones16)
            for idx, a in zip(idxs, avs):
                plsc.addupdate_scatter(hist, [idx + 2 * L * K], a)
            return carry
        return ibody

    ibodies = [make_ibody(xbuf, tbuf), make_ibody(xbuf2, tbuf2)]
    xbufs, tbufs = [xbuf, xbuf2], [tbuf, tbuf2]
    semxs, semts = [semx0, semx1], [semt0, semt1]
    nch = QUART // CHUNK
    rows_per_chunk = CHUNK // 512
    row0 = q * (QUART // 512)
    handles = [None, None]

    def issue(ch):
        rr = pl.multiple_of(row0 + ch * rows_per_chunk, rows_per_chunk)
        k = ch % 2
        handles[k] = (
            pltpu.async_copy(x_hbm.at[img, pl.ds(rr, rows_per_chunk)], xbufs[k], semxs[k]),
            pltpu.async_copy(t_hbm.at[img, pl.ds(rr, rows_per_chunk)], tbufs[k], semts[k]),
        )

    issue(0)

    def zbody(i, carry):
        hist[pl.ds(i * 16, 16)] = zero16
        return carry

    lax.fori_loop(0, HSIZE // 16, zbody, 0)

    for ch in range(nch):
        if ch + 1 < nch:
            issue(ch + 1)
        hx, ht = handles[ch % 2]
        hx.wait()
        ht.wait()
        lax.fori_loop(0, CHUNK // (G * 16), ibodies[ch % 2], 0)

    def rbody(j, carry):
        r = j // (K // 16)
        cc = j % (K // 16)
        vs = [hist[pl.ds(r * (L * K) + l * K + cc * 16, 16)] for l in range(L)]
        while len(vs) > 1:
            vs = [a + b for a, b in zip(vs[0::2], vs[1::2])]
        hist2[pl.ds(j * 16, 16)] = vs[0]
        return carry

    lax.fori_loop(0, ROWS // 16, rbody, 0)

    pltpu.sync_copy(hist2, shared.at[s])
    plsc.subcore_barrier()

    @pl.when(q == 0)
    def _leader():
        for r in range(1, 4):
            pltpu.sync_copy(shared.at[s + r], tmp2)

            def abody(j, carry):
                hist2[pl.ds(j * 16, 16)] = (
                    hist2[pl.ds(j * 16, 16)] + tmp2[pl.ds(j * 16, 16)]
                )
                return carry

            lax.fori_loop(0, ROWS // 16, abody, 0)

        def pbody(j, carry):
            carry_n, carry_a, acc_num, acc_den = carry
            o = j * 16
            om = K - 16 - o
            n0 = lax.rev(hist2[pl.ds(om, 16)], (0,))
            n1 = hist2[pl.ds(K + o, 16)]
            s0 = lax.rev(hist2[pl.ds(2 * K + om, 16)], (0,))
            s1 = hist2[pl.ds(3 * K + o, 16)]
            n = n0 + n1
            sv = s0 + s1
            csn = plsc.cumsum(n)
            csa = plsc.cumsum(sv)
            ne = carry_n + csn - n
            ae = carry_a + csa - sv
            acc_den = acc_den + n1 * (ne + 0.5 * (n + 1.0))
            acc_num = acc_num + n1 * (ae + 0.5 * sv) + 0.5 * s1
            return (carry_n + jnp.sum(n), carry_a + jnp.sum(sv),
                    acc_num, acc_den)

        carry_n, carry_a, acc_num, acc_den = lax.fori_loop(
            0, K // 16, pbody,
            (jnp.float32(0.0), jnp.float32(0.0), zero16, zero16))
        numv = zero16 + jnp.sum(acc_num)
        denv = zero16 + jnp.sum(acc_den)
        safev = jnp.where(denv == 0.0, 1.0, denv)
        obuf[...] = jnp.where(denv == 0.0, 0.0, numv / safev)
        pltpu.sync_copy(obuf, out_hbm.at[img])


_mesh = plsc.VectorSubcoreMesh(core_axis_name="c", subcore_axis_name="s")

_lovasz_sc = functools.partial(
    pl.kernel,
    out_type=jax.ShapeDtypeStruct((NIMG, L), jnp.float32),
    mesh=_mesh,
    compiler_params=pltpu.CompilerParams(
        needs_layout_passes=False, use_tc_tiling_on_sc=True),
    scratch_types=[
        pltpu.VMEM((CHUNK // 512, 512), jnp.float32),
        pltpu.VMEM((CHUNK // 512, 512), jnp.int32),
        pltpu.VMEM((CHUNK // 512, 512), jnp.float32),
        pltpu.VMEM((CHUNK // 512, 512), jnp.int32),
        pltpu.VMEM((HSIZE,), jnp.float32),
        pltpu.VMEM((ROWS,), jnp.float32),
        pltpu.VMEM((ROWS,), jnp.float32),
        pltpu.VMEM((L,), jnp.float32),
        pltpu.VMEM_SHARED((L, ROWS), jnp.float32),
        pltpu.SemaphoreType.DMA,
        pltpu.SemaphoreType.DMA,
        pltpu.SemaphoreType.DMA,
        pltpu.SemaphoreType.DMA,
    ],
)(_body)


def kernel(inputs, targets):
    losses = _lovasz_sc(inputs, targets)
    return jnp.mean(losses[:, 0])


# speedup vs baseline: 1.4002x; 1.4002x over previous
import functools

import jax
import jax.numpy as jnp
from jax import lax
from jax.experimental import pallas as pl
from jax.experimental.pallas import tpu as pltpu
from jax.experimental.pallas import tpu_sc as plsc

K = 1024
L = 16
NIMG = 8
NPIX = 512 * 512
QUART = NPIX // 4
CHUNK = 8192
SCALE = 2048.0
HSIZE = 2 * L * K
ROWS = 2 * K


def _body(x_hbm, t_hbm, out_hbm, xbuf, tbuf, xbuf2, tbuf2, hist, hist2, tmp2,
          obuf, shared, semx0, semt0, semx1, semt1):
    c = lax.axis_index("c")
    s = lax.axis_index("s")
    img = c * 4 + s // 4
    q = s % 4

    lane_off = lax.iota(jnp.int32, 16) * K
    zero16 = jnp.zeros((16,), jnp.float32)
    ones16 = jnp.ones((16,), jnp.float32)


    G = 16
    amax = jnp.float32(1.0 - 2e-7)

    def make_ibody(xb, tb):
        ipr = 512 // (G * 16)

        def ibody(i, carry):
            row = i // ipr
            cb = (i % ipr) * (G * 16)
            xs = [xb[row, pl.ds(cb + g * 16, 16)] for g in range(G)]
            ts = [tb[row, pl.ds(cb + g * 16, 16)] for g in range(G)]
            es = [jnp.exp(-x) for x in xs]
            ps = [1.0 / (1.0 + e) for e in es]
            us = [p + p - 1.0 for p in ps]
            avs = [jnp.minimum(jnp.abs(u), amax) for u in us]
            bs = [((u + 1.0) * (K * 0.5)).astype(jnp.int32) for u in us]
            bs = [jnp.minimum(b, K - 1) for b in bs]
            idxs = [(t << 14) | lane_off | b for t, b in zip(ts, bs)]
            for idx, a in zip(idxs, avs):
                plsc.addupdate_scatter(hist, [idx], a + SCALE)
            return carry
        return ibody

    ibodies = [make_ibody(xbuf, tbuf), make_ibody(xbuf2, tbuf2)]
    xbufs, tbufs = [xbuf, xbuf2], [tbuf, tbuf2]
    semxs, semts = [semx0, semx1], [semt0, semt1]
    nch = QUART // CHUNK
    rows_per_chunk = CHUNK // 512
    row0 = q * (QUART // 512)
    handles = [None, None]

    def issue(ch):
        rr = pl.multiple_of(row0 + ch * rows_per_chunk, rows_per_chunk)
        k = ch % 2
        handles[k] = (
            pltpu.async_copy(x_hbm.at[img, pl.ds(rr, rows_per_chunk)], xbufs[k], semxs[k]),
            pltpu.async_copy(t_hbm.at[img, pl.ds(rr, rows_per_chunk)], tbufs[k], semts[k]),
        )

    issue(0)

    def zbody(i, carry):
        hist[pl.ds(i * 16, 16)] = zero16
        return carry

    lax.fori_loop(0, HSIZE // 16, zbody, 0)

    for ch in range(nch):
        if ch + 1 < nch:
            issue(ch + 1)
        hx, ht = handles[ch % 2]
        hx.wait()
        ht.wait()
        lax.fori_loop(0, CHUNK // (G * 16), ibodies[ch % 2], 0)

    def rbody(j, carry):
        r = j // (K // 16)
        cc = j % (K // 16)
        vs = [hist[pl.ds(r * (L * K) + l * K + cc * 16, 16)] for l in range(L)]
        while len(vs) > 1:
            vs = [a + b for a, b in zip(vs[0::2], vs[1::2])]
        hist2[pl.ds(j * 16, 16)] = vs[0]
        return carry

    lax.fori_loop(0, ROWS // 16, rbody, 0)

    pltpu.sync_copy(hist2, shared.at[s])
    plsc.subcore_barrier()

    @pl.when(q == 0)
    def _leader():
        for r in range(1, 4):
            pltpu.sync_copy(shared.at[s + r], tmp2)

            def abody(j, carry):
                hist2[pl.ds(j * 16, 16)] = (
                    hist2[pl.ds(j * 16, 16)] + tmp2[pl.ds(j * 16, 16)]
                )
                return carry

            lax.fori_loop(0, ROWS // 16, abody, 0)

        inv_scale = jnp.float32(1.0 / SCALE)

        def pbody(j, carry):
            carry_n, carry_a, acc_num, acc_den = carry
            o = j * 16
            om = K - 16 - o
            h0 = lax.rev(hist2[pl.ds(om, 16)], (0,))
            h1 = hist2[pl.ds(K + o, 16)]
            n0 = (h0 * inv_scale).astype(jnp.int32).astype(jnp.float32)
            n1 = (h1 * inv_scale).astype(jnp.int32).astype(jnp.float32)
            s0 = h0 - SCALE * n0
            s1 = h1 - SCALE * n1
            n = n0 + n1
            sv = s0 + s1
            csn = plsc.cumsum(n)
            csa = plsc.cumsum(sv)
            ne = carry_n + csn - n
            ae = carry_a + csa - sv
            acc_den = acc_den + n1 * (ne + 0.5 * (n + 1.0))
            acc_num = acc_num + n1 * (ae + 0.5 * sv) + 0.5 * s1
            return (carry_n + jnp.sum(n), carry_a + jnp.sum(sv),
                    acc_num, acc_den)

        carry_n, carry_a, acc_num, acc_den = lax.fori_loop(
            0, K // 16, pbody,
            (jnp.float32(0.0), jnp.float32(0.0), zero16, zero16))
        numv = zero16 + jnp.sum(acc_num)
        denv = zero16 + jnp.sum(acc_den)
        safev = jnp.where(denv == 0.0, 1.0, denv)
        obuf[...] = jnp.where(denv == 0.0, 0.0, numv / safev)
        pltpu.sync_copy(obuf, out_hbm.at[img])


_mesh = plsc.VectorSubcoreMesh(core_axis_name="c", subcore_axis_name="s")

_lovasz_sc = functools.partial(
    pl.kernel,
    out_type=jax.ShapeDtypeStruct((NIMG, L), jnp.float32),
    mesh=_mesh,
    compiler_params=pltpu.CompilerParams(
        needs_layout_passes=False, use_tc_tiling_on_sc=True),
    scratch_types=[
        pltpu.VMEM((CHUNK // 512, 512), jnp.float32),
        pltpu.VMEM((CHUNK // 512, 512), jnp.int32),
        pltpu.VMEM((CHUNK // 512, 512), jnp.float32),
        pltpu.VMEM((CHUNK // 512, 512), jnp.int32),
        pltpu.VMEM((HSIZE,), jnp.float32),
        pltpu.VMEM((ROWS,), jnp.float32),
        pltpu.VMEM((ROWS,), jnp.float32),
        pltpu.VMEM((L,), jnp.float32),
        pltpu.VMEM_SHARED((L, ROWS), jnp.float32),
        pltpu.SemaphoreType.DMA,
        pltpu.SemaphoreType.DMA,
        pltpu.SemaphoreType.DMA,
        pltpu.SemaphoreType.DMA,
    ],
)(_body)


def kernel(inputs, targets):
    losses = _lovasz_sc(inputs, targets)
    return jnp.mean(losses[:, 0])


# speedup vs baseline: 1.4680x; 1.0484x over previous
import functools

import jax
import jax.numpy as jnp
from jax import lax
from jax.experimental import pallas as pl
from jax.experimental.pallas import tpu as pltpu
from jax.experimental.pallas import tpu_sc as plsc

K = 1024
L = 16
NIMG = 8
NPIX = 512 * 512
QUART = NPIX // 4
CHUNK = 8192
SCALE = 2048.0
HSIZE = 2 * L * K
ROWS = 2 * K


def _body(x_hbm, t_hbm, out_hbm, xbuf, tbuf, xbuf2, tbuf2, hist, hist2, tmp2,
          obuf, shared, semx0, semt0, semx1, semt1):
    c = lax.axis_index("c")
    s = lax.axis_index("s")
    img = c * 4 + s // 4
    q = s % 4

    lane_off = lax.iota(jnp.int32, 16) * K
    zero16 = jnp.zeros((16,), jnp.float32)
    ones16 = jnp.ones((16,), jnp.float32)


    G = 16
    amax = jnp.float32(1.0 - 2e-7)

    def make_ibody(xb, tb):
        ipr = 512 // (G * 16)

        def ibody(i, carry):
            row = i // ipr
            cb = (i % ipr) * (G * 16)
            xs = [xb[row, pl.ds(cb + g * 16, 16)] for g in range(G)]
            ts = [tb[row, pl.ds(cb + g * 16, 16)] for g in range(G)]
            es = [jnp.exp(-x) for x in xs]
            ps = [1.0 / (1.0 + e) for e in es]
            us = [p + p - 1.0 for p in ps]
            avs = [jnp.minimum(jnp.abs(u), amax) for u in us]
            bs = [(p * jnp.float32(K)).astype(jnp.int32) for p in ps]
            bs = [jnp.minimum(b, K - 1) for b in bs]
            idxs = [(t << 14) | lane_off | b for t, b in zip(ts, bs)]
            for idx, a in zip(idxs, avs):
                plsc.addupdate_scatter(hist, [idx], a + SCALE)
            return carry
        return ibody

    ibodies = [make_ibody(xbuf, tbuf), make_ibody(xbuf2, tbuf2)]
    xbufs, tbufs = [xbuf, xbuf2], [tbuf, tbuf2]
    semxs, semts = [semx0, semx1], [semt0, semt1]
    nch = QUART // CHUNK
    rows_per_chunk = CHUNK // 512
    row0 = q * (QUART // 512)

    def issue(ch, k):
        rr = pl.multiple_of(row0 + ch * rows_per_chunk, rows_per_chunk)
        pltpu.async_copy(x_hbm.at[img, pl.ds(rr, rows_per_chunk)], xbufs[k], semxs[k])
        pltpu.async_copy(t_hbm.at[img, pl.ds(rr, rows_per_chunk)], tbufs[k], semts[k])

    def wait(k):
        pltpu.make_async_copy(
            x_hbm.at[0, pl.ds(0, rows_per_chunk)], xbufs[k], semxs[k]).wait()
        pltpu.make_async_copy(
            t_hbm.at[0, pl.ds(0, rows_per_chunk)], tbufs[k], semts[k]).wait()

    issue(0, 0)
    issue(1, 1)

    def zbody(i, carry):
        hist[pl.ds(i * 16, 16)] = zero16
        return carry

    lax.fori_loop(0, HSIZE // 16, zbody, 0)

    def pair_body(kp, carry):
        for k in range(2):
            ch = kp * 2 + k
            wait(k)
            lax.fori_loop(0, CHUNK // (G * 16), ibodies[k], 0)

            @pl.when(ch + 2 < nch)
            def _():
                issue(ch + 2, k)
        return carry

    lax.fori_loop(0, nch // 2, pair_body, 0)

    def rbody(j, carry):
        r = j // (K // 16)
        cc = j % (K // 16)
        vs = [hist[pl.ds(r * (L * K) + l * K + cc * 16, 16)] for l in range(L)]
        while len(vs) > 1:
            vs = [a + b for a, b in zip(vs[0::2], vs[1::2])]
        hist2[pl.ds(j * 16, 16)] = vs[0]
        return carry

    lax.fori_loop(0, ROWS // 16, rbody, 0)

    pltpu.sync_copy(hist2, shared.at[s])
    plsc.subcore_barrier()

    @pl.when(q == 0)
    def _leader():
        for r in range(1, 4):
            pltpu.sync_copy(shared.at[s + r], tmp2)

            def abody(j, carry):
                hist2[pl.ds(j * 16, 16)] = (
                    hist2[pl.ds(j * 16, 16)] + tmp2[pl.ds(j * 16, 16)]
                )
                return carry

            lax.fori_loop(0, ROWS // 16, abody, 0)

        inv_scale = jnp.float32(1.0 / SCALE)

        def pbody(j, carry):
            carry_n, carry_a, acc_num, acc_den = carry
            o = j * 16
            om = K - 16 - o
            h0 = lax.rev(hist2[pl.ds(om, 16)], (0,))
            h1 = hist2[pl.ds(K + o, 16)]
            n0 = (h0 * inv_scale).astype(jnp.int32).astype(jnp.float32)
            n1 = (h1 * inv_scale).astype(jnp.int32).astype(jnp.float32)
            s0 = h0 - SCALE * n0
            s1 = h1 - SCALE * n1
            n = n0 + n1
            sv = s0 + s1
            csn = plsc.cumsum(n)
            csa = plsc.cumsum(sv)
            ne = carry_n + csn - n
            ae = carry_a + csa - sv
            acc_den = acc_den + n1 * (ne + 0.5 * (n + 1.0))
            acc_num = acc_num + n1 * (ae + 0.5 * sv) + 0.5 * s1
            return (carry_n + jnp.sum(n), carry_a + jnp.sum(sv),
                    acc_num, acc_den)

        carry_n, carry_a, acc_num, acc_den = lax.fori_loop(
            0, K // 16, pbody,
            (jnp.float32(0.0), jnp.float32(0.0), zero16, zero16))
        numv = zero16 + jnp.sum(acc_num)
        denv = zero16 + jnp.sum(acc_den)
        safev = jnp.where(denv == 0.0, 1.0, denv)
        obuf[...] = jnp.where(denv == 0.0, 0.0, numv / safev)
        pltpu.sync_copy(obuf, out_hbm.at[img])


_mesh = plsc.VectorSubcoreMesh(core_axis_name="c", subcore_axis_name="s")

_lovasz_sc = functools.partial(
    pl.kernel,
    out_type=jax.ShapeDtypeStruct((NIMG, L), jnp.float32),
    mesh=_mesh,
    compiler_params=pltpu.CompilerParams(
        needs_layout_passes=False, use_tc_tiling_on_sc=True),
    scratch_types=[
        pltpu.VMEM((CHUNK // 512, 512), jnp.float32),
        pltpu.VMEM((CHUNK // 512, 512), jnp.int32),
        pltpu.VMEM((CHUNK // 512, 512), jnp.float32),
        pltpu.VMEM((CHUNK // 512, 512), jnp.int32),
        pltpu.VMEM((HSIZE,), jnp.float32),
        pltpu.VMEM((ROWS,), jnp.float32),
        pltpu.VMEM((ROWS,), jnp.float32),
        pltpu.VMEM((L,), jnp.float32),
        pltpu.VMEM_SHARED((L, ROWS), jnp.float32),
        pltpu.SemaphoreType.DMA,
        pltpu.SemaphoreType.DMA,
        pltpu.SemaphoreType.DMA,
        pltpu.SemaphoreType.DMA,
    ],
)(_body)


def kernel(inputs, targets):
    losses = _lovasz_sc(inputs, targets)
    return jnp.mean(losses[:, 0])
